# baseline (speedup 1.0000x reference)
import functools

import jax
import jax.numpy as jnp
from jax import lax
from jax.experimental import pallas as pl
from jax.experimental.pallas import tpu as pltpu
from jax.experimental.pallas import tpu_sc as plsc

_FIELDS = 26
_K = 16


def _sc_gather(feature_flat, v_table, w16_table):
    n = feature_flat.shape[0]
    info = plsc.get_sparse_core_info()
    nw = info.num_cores * info.num_subcores
    per_w = n // nw
    chunk = 1024
    n_chunks = per_w // chunk
    mesh = plsc.VectorSubcoreMesh(core_axis_name="c", subcore_axis_name="s")

    @functools.partial(
        pl.kernel,
        mesh=mesh,
        compiler_params=pltpu.CompilerParams(
            use_tc_tiling_on_sc=False, needs_layout_passes=False),
        out_type=(
            jax.ShapeDtypeStruct((n, _K), jnp.float32),
            jax.ShapeDtypeStruct((n,), jnp.float32),
        ),
        scratch_types=[
            pltpu.VMEM((chunk,), jnp.int32),
            pltpu.VMEM((chunk,), jnp.int32),
            pltpu.VMEM((chunk,), jnp.int32),
            pltpu.VMEM((chunk, _K), jnp.float32),
            pltpu.VMEM((chunk, _K), jnp.float32),
            pltpu.VMEM((chunk,), jnp.float32),
            pltpu.SemaphoreType.DMA,
            pltpu.SemaphoreType.DMA,
        ],
    )
    def k(feat_hbm, vtab_hbm, w16_hbm, outv_hbm, outw_hbm,
          idx_v, rowids_v, vrow_v, rows_v, w16rows_v, wvals_v, sem_v, sem_w):
        wid = lax.axis_index("s") * info.num_cores + lax.axis_index("c")
        base = wid * per_w

        def body(i, carry):
            off = base + i * chunk
            pltpu.sync_copy(feat_hbm.at[pl.ds(off, chunk)], idx_v)

            def mk_rowids(j, c):
                v = idx_v[pl.ds(j * 16, 16)]
                rowids_v[pl.ds(j * 16, 16)] = lax.shift_right_logical(v, 4)
                vrow_v[pl.ds(j * 16, 16)] = lax.shift_left(v, 3)
                return c
            lax.fori_loop(0, chunk // 16, mk_rowids, 0)

            cp_v = pltpu.async_copy(vtab_hbm.at[vrow_v], rows_v, sem_v)
            cp_w = pltpu.async_copy(w16_hbm.at[rowids_v], w16rows_v, sem_w)
            cp_v.wait()
            pltpu.sync_copy(rows_v, outv_hbm.at[pl.ds(off, chunk)])
            cp_w.wait()

            def extract(j, c):
                lane = idx_v[pl.ds(j * 16, 16)] & 15
                rowv = lax.iota(jnp.int32, 16) + j * 16
                wvals_v[pl.ds(j * 16, 16)] = plsc.load_gather(
                    w16rows_v, [rowv, lane])
                return c
            lax.fori_loop(0, chunk // 16, extract, 0)

            pltpu.sync_copy(wvals_v, outw_hbm.at[pl.ds(off, chunk)])
            return carry

        lax.fori_loop(0, n_chunks, body, 0)

    return k(feature_flat, v_table, w16_table)


def _padlin_body(vt_ref, out_ref):
    out_ref[:, 0:_K] = vt_ref[...].T


def _tc_padlin(vt):
    rows, = vt.shape[1:]
    blk = 4096
    grid = (pl.cdiv(rows, blk),)
    return pl.pallas_call(
        _padlin_body,
        grid=grid,
        in_specs=[pl.BlockSpec((_K, blk), lambda i: (0, i))],
        out_specs=pl.BlockSpec((blk, 128), lambda i: (i, 0)),
        out_shape=jax.ShapeDtypeStruct((rows, 128), jnp.float32),
    )(vt)


def _tc_body(v_ref, wg_ref, w1_ref, b1_ref, w2_ref, b2_ref, w3_ref, b3_ref,
             wo_ref, bo_ref, out_ref):
    emb = v_ref[...]
    blk = emb.shape[0]
    d = emb.shape[1]
    sel = (lax.broadcasted_iota(jnp.int32, (d, _K), 0) % _K
           == lax.broadcasted_iota(jnp.int32, (d, _K), 1))
    a_f32 = sel.astype(jnp.float32)
    dn0 = (((1,), (0,)), ((), ()))
    s = lax.dot_general(emb, a_f32, dn0, preferred_element_type=jnp.float32)
    ss = lax.dot_general(emb * emb, a_f32, dn0,
                         preferred_element_type=jnp.float32)
    fm = 0.5 * jnp.sum(s * s - ss, axis=1, keepdims=True)
    wsum = jnp.sum(wg_ref[...], axis=1, keepdims=True)

    dn = (((1,), (1,)), ((), ()))
    h = jnp.maximum(
        lax.dot_general(emb, w1_ref[...], dn, preferred_element_type=jnp.float32)
        + b1_ref[...], 0.0)
    h = jnp.maximum(
        lax.dot_general(h, w2_ref[...], dn, preferred_element_type=jnp.float32)
        + b2_ref[...], 0.0)
    h = jnp.maximum(
        lax.dot_general(h, w3_ref[...], dn, preferred_element_type=jnp.float32)
        + b3_ref[...], 0.0)
    dnn = lax.dot_general(h, wo_ref[...], dn, preferred_element_type=jnp.float32)
    out_ref[...] = fm + wsum + dnn + bo_ref[...]


def _tc_compute(V2, Wg2, W1, b1, W2, b2, W3, b3, Wout, bout, *, interpret=False):
    batch, d = V2.shape
    blk = 1024
    grid = (batch // blk,)
    full = lambda shape: pl.BlockSpec(shape, lambda i: (0, 0))
    return pl.pallas_call(
        _tc_body,
        grid=grid,
        in_specs=[
            pl.BlockSpec((blk, d), lambda i: (i, 0)),
            pl.BlockSpec((blk, _FIELDS), lambda i: (i, 0)),
            full(W1.shape), full(b1.shape),
            full(W2.shape), full(b2.shape),
            full(W3.shape), full(b3.shape),
            full(Wout.shape), full(bout.shape),
        ],
        out_specs=pl.BlockSpec((blk, 1), lambda i: (i, 0)),
        out_shape=jax.ShapeDtypeStruct((batch, 1), jnp.float32),
        interpret=interpret,
    )(V2, Wg2, W1, b1, W2, b2, W3, b3, Wout, bout)


def kernel(feature, w_table, v_table, W1, b1, W2, b2, W3, b3, Wout, bout):
    batch, fields = feature.shape
    flat_idx = feature.reshape(-1).astype(jnp.int32)
    w16 = lax.optimization_barrier(w_table.reshape(-1)).reshape(-1, _K)
    v_pad = _tc_padlin(v_table.T).reshape(-1, _K)
    V, Wg = _sc_gather(flat_idx, v_pad, w16)
    V2 = V.reshape(batch, fields * _K)
    Wg2 = Wg.reshape(batch, fields)
    return _tc_compute(V2, Wg2, W1, b1.reshape(1, -1), W2, b2.reshape(1, -1),
                       W3, b3.reshape(1, -1), Wout, bout.reshape(1, 1))


# speedup vs baseline: 1.1491x; 1.1491x over previous
import functools

import jax
import jax.numpy as jnp
from jax import lax
from jax.experimental import pallas as pl
from jax.experimental.pallas import tpu as pltpu
from jax.experimental.pallas import tpu_sc as plsc

_FIELDS = 26
_K = 16


def _sc_gather(feature_flat, v_table, w16_table):
    n = feature_flat.shape[0]
    info = plsc.get_sparse_core_info()
    nw = info.num_cores * info.num_subcores
    per_w = n // nw
    chunk = 1024
    n_chunks = per_w // chunk
    mesh = plsc.VectorSubcoreMesh(core_axis_name="c", subcore_axis_name="s")

    @functools.partial(
        pl.kernel,
        mesh=mesh,
        compiler_params=pltpu.CompilerParams(
            use_tc_tiling_on_sc=False, needs_layout_passes=False),
        out_type=(
            jax.ShapeDtypeStruct((n, _K), jnp.float32),
            jax.ShapeDtypeStruct((n,), jnp.float32),
        ),
        scratch_types=[
            pltpu.VMEM((chunk,), jnp.int32),
            pltpu.VMEM((chunk,), jnp.int32),
            pltpu.VMEM((chunk,), jnp.int32),
            pltpu.VMEM((chunk, _K), jnp.float32),
            pltpu.VMEM((chunk, _K), jnp.float32),
            pltpu.VMEM((chunk,), jnp.float32),
            pltpu.SemaphoreType.DMA,
            pltpu.SemaphoreType.DMA,
        ],
    )
    def k(feat_hbm, vtab_hbm, w16_hbm, outv_hbm, outw_hbm,
          idx_v, rowids_v, vrow_v, rows_v, w16rows_v, wvals_v, sem_v, sem_w):
        wid = lax.axis_index("s") * info.num_cores + lax.axis_index("c")
        base = wid * per_w

        def body(i, carry):
            off = base + i * chunk
            pltpu.sync_copy(feat_hbm.at[pl.ds(off, chunk)], idx_v)

            def mk_rowids(j, c):
                v = idx_v[pl.ds(j * 16, 16)]
                rowids_v[pl.ds(j * 16, 16)] = lax.shift_right_logical(v, 4)
                vrow_v[pl.ds(j * 16, 16)] = lax.shift_left(v, 3)
                return c
            lax.fori_loop(0, chunk // 16, mk_rowids, 0)

            cp_v = pltpu.async_copy(vtab_hbm.at[vrow_v], rows_v, sem_v)
            cp_w = pltpu.async_copy(w16_hbm.at[rowids_v], w16rows_v, sem_w)
            cp_v.wait()
            pltpu.sync_copy(rows_v, outv_hbm.at[pl.ds(off, chunk)])
            cp_w.wait()

            def extract(j, c):
                lane = idx_v[pl.ds(j * 16, 16)] & 15
                rowv = lax.iota(jnp.int32, 16) + j * 16
                wvals_v[pl.ds(j * 16, 16)] = plsc.load_gather(
                    w16rows_v, [rowv, lane])
                return c
            lax.fori_loop(0, chunk // 16, extract, 0)

            pltpu.sync_copy(wvals_v, outw_hbm.at[pl.ds(off, chunk)])
            return carry

        lax.fori_loop(0, n_chunks, body, 0)

    return k(feature_flat, v_table, w16_table)


def _padlin_body(vt_ref, out_ref):
    out_ref[:, 0:_K] = vt_ref[...].T


def _tc_padlin(vt):
    rows, = vt.shape[1:]
    blk = 8192
    grid = (pl.cdiv(rows, blk),)
    return pl.pallas_call(
        _padlin_body,
        grid=grid,
        in_specs=[pl.BlockSpec((_K, blk), lambda i: (0, i))],
        out_specs=pl.BlockSpec((blk, 128), lambda i: (i, 0)),
        out_shape=jax.ShapeDtypeStruct((rows, 128), jnp.float32),
    )(vt)


def _tc_body(v_ref, wg_ref, w1_ref, b1_ref, w2_ref, b2_ref, w3_ref, b3_ref,
             wo_ref, bo_ref, out_ref):
    emb = v_ref[...]
    blk = emb.shape[0]
    d = emb.shape[1]
    sel = (lax.broadcasted_iota(jnp.int32, (d, _K), 0) % _K
           == lax.broadcasted_iota(jnp.int32, (d, _K), 1))
    a_f32 = sel.astype(jnp.float32)
    dn0 = (((1,), (0,)), ((), ()))
    s = lax.dot_general(emb, a_f32, dn0, preferred_element_type=jnp.float32)
    ss = lax.dot_general(emb * emb, a_f32, dn0,
                         preferred_element_type=jnp.float32)
    fm = 0.5 * jnp.sum(s * s - ss, axis=1, keepdims=True)
    wsum = jnp.sum(wg_ref[...], axis=1, keepdims=True)

    dn = (((1,), (1,)), ((), ()))
    h = jnp.maximum(
        lax.dot_general(emb, w1_ref[...], dn, preferred_element_type=jnp.float32)
        + b1_ref[...], 0.0)
    h = jnp.maximum(
        lax.dot_general(h, w2_ref[...], dn, preferred_element_type=jnp.float32)
        + b2_ref[...], 0.0)
    h = jnp.maximum(
        lax.dot_general(h, w3_ref[...], dn, preferred_element_type=jnp.float32)
        + b3_ref[...], 0.0)
    dnn = lax.dot_general(h, wo_ref[...], dn, preferred_element_type=jnp.float32)
    out_ref[...] = fm + wsum + dnn + bo_ref[...]


def _tc_compute(V2, Wg2, W1, b1, W2, b2, W3, b3, Wout, bout, *, interpret=False):
    batch, d = V2.shape
    blk = 1024
    grid = (batch // blk,)
    full = lambda shape: pl.BlockSpec(shape, lambda i: (0, 0))
    return pl.pallas_call(
        _tc_body,
        grid=grid,
        in_specs=[
            pl.BlockSpec((blk, d), lambda i: (i, 0)),
            pl.BlockSpec((blk, _FIELDS), lambda i: (i, 0)),
            full(W1.shape), full(b1.shape),
            full(W2.shape), full(b2.shape),
            full(W3.shape), full(b3.shape),
            full(Wout.shape), full(bout.shape),
        ],
        out_specs=pl.BlockSpec((blk, 1), lambda i: (i, 0)),
        out_shape=jax.ShapeDtypeStruct((batch, 1), jnp.float32),
        interpret=interpret,
    )(V2, Wg2, W1, b1, W2, b2, W3, b3, Wout, bout)


def kernel(feature, w_table, v_table, W1, b1, W2, b2, W3, b3, Wout, bout):
    batch, fields = feature.shape
    flat_idx = feature.reshape(-1).astype(jnp.int32)
    w16 = lax.optimization_barrier(w_table.reshape(-1)).reshape(-1, _K)
    v_pad = _tc_padlin(v_table.T).reshape(-1, _K)
    V, Wg = _sc_gather(flat_idx, v_pad, w16)
    V2 = V.reshape(batch, fields * _K)
    Wg2 = Wg.reshape(batch, fields)
    return _tc_compute(V2, Wg2, W1, b1.reshape(1, -1), W2, b2.reshape(1, -1),
                       W3, b3.reshape(1, -1), Wout, bout.reshape(1, 1))
